# baseline (speedup 1.0000x reference)
import functools

import jax
import jax.numpy as jnp
from jax import lax
from jax.experimental import pallas as pl
from jax.experimental.pallas import tpu as pltpu
from jax.experimental.pallas import tpu_sc as plsc

D_VOCAB = 1000000
D_MODEL = 64
B_TOTAL = 4096 * 200

_info = plsc.get_sparse_core_info()
_NC, _NS = _info.num_cores, _info.num_subcores
_NW = _NC * _NS

_PER_W = B_TOTAL // _NW
_CHUNK = 128


@functools.partial(
    pl.kernel,
    out_type=jax.ShapeDtypeStruct((B_TOTAL, D_MODEL), jnp.float32),
    mesh=plsc.VectorSubcoreMesh(core_axis_name="c", subcore_axis_name="s"),
    compiler_params=pltpu.CompilerParams(use_tc_tiling_on_sc=False),
    scratch_types=[
        pltpu.VMEM((_CHUNK,), jnp.int32),
        pltpu.VMEM((_CHUNK, D_MODEL), jnp.float32),
        pltpu.SemaphoreType.DMA,
    ],
)
def _embed_sc(x_hbm, w_hbm, out_hbm, idx_v, rows_v, sem):
    wid = lax.axis_index("s") * _NC + lax.axis_index("c")
    w_base = wid * _PER_W

    def body(c, carry):
        base = w_base + c * _CHUNK
        pltpu.sync_copy(x_hbm.at[pl.ds(base, _CHUNK)], idx_v)
        pltpu.async_copy(w_hbm.at[idx_v], rows_v, sem).wait()
        pltpu.sync_copy(rows_v, out_hbm.at[pl.ds(base, _CHUNK)])
        return carry

    lax.fori_loop(0, _PER_W // _CHUNK, body, 0)


def kernel(x, weight):
    flat = _embed_sc(x.reshape(-1), weight)
    return flat.reshape(x.shape[0], x.shape[1], D_MODEL)


# speedup vs baseline: 1.1843x; 1.1843x over previous
import functools

import jax
import jax.numpy as jnp
from jax import lax
from jax.experimental import pallas as pl
from jax.experimental.pallas import tpu as pltpu
from jax.experimental.pallas import tpu_sc as plsc

D_VOCAB = 1000000
D_MODEL = 64
B_TOTAL = 4096 * 200

_info = plsc.get_sparse_core_info()
_NC, _NS = _info.num_cores, _info.num_subcores
_NW = _NC * _NS

_PER_W = B_TOTAL // _NW
_SUB = 128
_CHUNK = 512
_NSUB = _CHUNK // _SUB
_NCH = _PER_W // _CHUNK
_NBUF = 2


@functools.partial(
    pl.kernel,
    out_type=jax.ShapeDtypeStruct((B_TOTAL, D_MODEL), jnp.float32),
    mesh=plsc.VectorSubcoreMesh(core_axis_name="c", subcore_axis_name="s"),
    compiler_params=pltpu.CompilerParams(use_tc_tiling_on_sc=False),
    scratch_types=[
        pltpu.VMEM((_NBUF, _CHUNK), jnp.int32),
        pltpu.VMEM((_NBUF, _CHUNK, D_MODEL), jnp.float32),
        pltpu.SemaphoreType.DMA,
        pltpu.SemaphoreType.DMA,
    ],
)
def _embed_sc(x_hbm, w_hbm, out_hbm, idx_v, rows_v, gsem0, gsem1):
    wid = lax.axis_index("s") * _NC + lax.axis_index("c")
    w_base = wid * _PER_W
    gsems = (gsem0, gsem1)

    def stage(c, b):
        base = w_base + c * _CHUNK
        pltpu.sync_copy(x_hbm.at[pl.ds(base, _CHUNK)], idx_v.at[b])
        for j in range(_NSUB):
            pltpu.async_copy(
                w_hbm.at[idx_v.at[b, pl.ds(j * _SUB, _SUB)]],
                rows_v.at[b, pl.ds(j * _SUB, _SUB)],
                gsems[b],
            )

    def drain_write(c, b):
        pltpu.make_async_copy(w_hbm.at[pl.ds(0, _CHUNK)], rows_v.at[b],
                              gsems[b]).wait()
        base = w_base + c * _CHUNK
        pltpu.sync_copy(rows_v.at[b], out_hbm.at[pl.ds(base, _CHUNK)])

    for b in range(_NBUF):
        stage(b, b)

    def body(i, carry):
        for b in range(_NBUF):
            c = i * _NBUF + b
            drain_write(c, b)

            @pl.when(c + _NBUF < _NCH)
            def _():
                stage(c + _NBUF, b)

        return carry

    lax.fori_loop(0, _NCH // _NBUF, body, 0)


def kernel(x, weight):
    flat = _embed_sc(x.reshape(-1), weight)
    return flat.reshape(x.shape[0], x.shape[1], D_MODEL)


# speedup vs baseline: 1.1976x; 1.0113x over previous
import functools

import jax
import jax.numpy as jnp
from jax import lax
from jax.experimental import pallas as pl
from jax.experimental.pallas import tpu as pltpu
from jax.experimental.pallas import tpu_sc as plsc

D_VOCAB = 1000000
D_MODEL = 64
B_TOTAL = 4096 * 200

_info = plsc.get_sparse_core_info()
_NC, _NS = _info.num_cores, _info.num_subcores
_NW = _NC * _NS

_PER_W = B_TOTAL // _NW
_SUB = 128
_CHUNK = 256
_NSUB = _CHUNK // _SUB
_NCH = _PER_W // _CHUNK
_NBUF = 4
_DIST = 2


@functools.partial(
    pl.kernel,
    out_type=jax.ShapeDtypeStruct((B_TOTAL, D_MODEL), jnp.float32),
    mesh=plsc.VectorSubcoreMesh(core_axis_name="c", subcore_axis_name="s"),
    compiler_params=pltpu.CompilerParams(use_tc_tiling_on_sc=False),
    scratch_types=[
        pltpu.VMEM((_PER_W,), jnp.int32),
        pltpu.VMEM((_NBUF, _CHUNK, D_MODEL), jnp.float32),
        [pltpu.SemaphoreType.DMA] * _NBUF,
        [pltpu.SemaphoreType.DMA] * _NBUF,
    ],
)
def _embed_sc(x_hbm, w_hbm, out_hbm, idx_v, rows_v, gsems, wsems):
    wid = lax.axis_index("s") * _NC + lax.axis_index("c")
    w_base = wid * _PER_W

    pltpu.sync_copy(x_hbm.at[pl.ds(w_base, _PER_W)], idx_v)

    def fire(c, b):
        for j in range(_NSUB):
            pltpu.async_copy(
                w_hbm.at[idx_v.at[pl.ds(c * _CHUNK + j * _SUB, _SUB)]],
                rows_v.at[b, pl.ds(j * _SUB, _SUB)],
                gsems[b],
            )

    def drain_gathers(b):
        pltpu.make_async_copy(w_hbm.at[pl.ds(0, _CHUNK)], rows_v.at[b],
                              gsems[b]).wait()

    def wait_wb(b):
        pltpu.make_async_copy(rows_v.at[b], out_hbm.at[pl.ds(0, _CHUNK)],
                              wsems[b]).wait()

    for k in range(_DIST):
        fire(k, k)

    def body(g, carry):
        for k in range(_NBUF):
            c = g * _NBUF + k
            drain_gathers(k)
            pltpu.async_copy(rows_v.at[k],
                             out_hbm.at[pl.ds(w_base + c * _CHUNK, _CHUNK)],
                             wsems[k])
            b2 = (k + _DIST) % _NBUF

            @pl.when(c >= _NBUF - _DIST)
            def _():
                wait_wb(b2)

            @pl.when(c + _DIST < _NCH)
            def _():
                fire(c + _DIST, b2)

        return carry

    lax.fori_loop(0, _NCH // _NBUF, body, 0)

    for k in range(_DIST):
        wait_wb((_NCH - _DIST + k) % _NBUF)


def kernel(x, weight):
    flat = _embed_sc(x.reshape(-1), weight)
    return flat.reshape(x.shape[0], x.shape[1], D_MODEL)
